# baseline (speedup 1.0000x reference)
import functools

import jax
import jax.numpy as jnp
import numpy as np
from jax import lax
from jax.experimental import pallas as pl
from jax.experimental.pallas import tpu as pltpu
from jax.experimental.pallas import tpu_sc as plsc

VOCAB = 1000000
DIM = 64
MAX_LEN = 256
B = 4096
S = 200

NUM_CORES = 2
NUM_SUBCORES = 16
NW = NUM_CORES * NUM_SUBCORES
SEQ_PER_W = B // NW
HALF = S // 2


def _sinusoidal_pe(max_len, dim):
    pos = np.arange(max_len, dtype=np.float32)[:, None]
    i = np.arange(0, dim, 2, dtype=np.float32)[None, :]
    angle = pos / np.power(10000.0, i / dim)
    pe = np.zeros((max_len, dim), dtype=np.float32)
    pe[:, 0::2] = np.sin(angle)
    pe[:, 1::2] = np.cos(angle)
    return pe


_PE = _sinusoidal_pe(MAX_LEN, DIM)[:S, :]


@functools.partial(
    pl.kernel,
    mesh=plsc.VectorSubcoreMesh(core_axis_name="c", subcore_axis_name="s"),
    out_type=jax.ShapeDtypeStruct((B * S, DIM), jnp.float32),
    compiler_params=pltpu.CompilerParams(use_tc_tiling_on_sc=False),
    scratch_types=[
        pltpu.VMEM((2, HALF), jnp.int32),
        pltpu.VMEM((S, DIM), jnp.float32),
        pltpu.VMEM((S, DIM), jnp.float32),
        pltpu.SemaphoreType.DMA,
    ],
)
def _emb(idx_hbm, table_hbm, pe_hbm, out_hbm, idx_v, rows_v, pe_v, sem):
    wid = lax.axis_index("s") * NUM_CORES + lax.axis_index("c")
    pltpu.sync_copy(pe_hbm, pe_v)

    def body(i, _):
        seq = wid * SEQ_PER_W + i
        pltpu.sync_copy(idx_hbm.at[pl.ds(seq * 2, 2)], idx_v)
        cp0 = pltpu.async_copy(table_hbm.at[idx_v.at[0]], rows_v.at[pl.ds(0, HALF)], sem)
        cp1 = pltpu.async_copy(table_hbm.at[idx_v.at[1]], rows_v.at[pl.ds(HALF, HALF)], sem)
        cp0.wait()
        cp1.wait()

        def add_body(r, _):
            for c4 in range(DIM // 16):
                sl = pl.ds(c4 * 16, 16)
                rows_v[r, sl] = rows_v[r, sl] + pe_v[r, sl]
            return 0

        lax.fori_loop(0, S, add_body, 0)
        pltpu.sync_copy(rows_v, out_hbm.at[pl.ds(seq * S, S)])
        return 0

    lax.fori_loop(0, SEQ_PER_W, body, 0)


def kernel(input, tok_table):
    idx = input.astype(jnp.int32).reshape(B * 2, HALF)
    out = _emb(idx, tok_table, jnp.asarray(_PE))
    return out.reshape(B, S, DIM)


# speedup vs baseline: 1.1966x; 1.1966x over previous
import functools

import jax
import jax.numpy as jnp
import numpy as np
from jax import lax
from jax.experimental import pallas as pl
from jax.experimental.pallas import tpu as pltpu
from jax.experimental.pallas import tpu_sc as plsc

VOCAB = 1000000
DIM = 64
MAX_LEN = 256
B = 4096
S = 200

NUM_CORES = 2
NUM_SUBCORES = 16
NW = NUM_CORES * NUM_SUBCORES
SEQ_PER_W = B // NW
HALF = S // 2
SEQ_PER_CHUNK = 2
ROWS_PER_CHUNK = SEQ_PER_CHUNK * S
CHUNKS = SEQ_PER_W // SEQ_PER_CHUNK
IDX_ROWS = SEQ_PER_W * 2
GATHERS = ROWS_PER_CHUNK // HALF


def _sinusoidal_pe(max_len, dim):
    pos = np.arange(max_len, dtype=np.float32)[:, None]
    i = np.arange(0, dim, 2, dtype=np.float32)[None, :]
    angle = pos / np.power(10000.0, i / dim)
    pe = np.zeros((max_len, dim), dtype=np.float32)
    pe[:, 0::2] = np.sin(angle)
    pe[:, 1::2] = np.cos(angle)
    return pe


_PE = _sinusoidal_pe(MAX_LEN, DIM)[:S, :]


@functools.partial(
    pl.kernel,
    mesh=plsc.VectorSubcoreMesh(core_axis_name="c", subcore_axis_name="s"),
    out_type=jax.ShapeDtypeStruct((B * S, DIM), jnp.float32),
    compiler_params=pltpu.CompilerParams(use_tc_tiling_on_sc=False),
    scratch_types=[
        pltpu.VMEM((IDX_ROWS, HALF), jnp.int32),
        pltpu.VMEM((ROWS_PER_CHUNK, DIM), jnp.float32),
        pltpu.VMEM((ROWS_PER_CHUNK, DIM), jnp.float32),
        pltpu.VMEM((S, DIM), jnp.float32),
        pltpu.SemaphoreType.DMA,
        pltpu.SemaphoreType.DMA,
        pltpu.SemaphoreType.DMA,
        pltpu.SemaphoreType.DMA,
    ],
)
def _emb(idx_hbm, table_hbm, pe_hbm, out_hbm,
         idx_v, rows0, rows1, pe_v, gsem0, gsem1, osem0, osem1):
    wid = lax.axis_index("s") * NUM_CORES + lax.axis_index("c")
    rows = (rows0, rows1)
    gsem = (gsem0, gsem1)
    osem = (osem0, osem1)

    pltpu.sync_copy(pe_hbm, pe_v)
    pltpu.sync_copy(idx_hbm.at[pl.ds(wid * IDX_ROWS, IDX_ROWS)], idx_v)
    chunk0 = wid * CHUNKS

    def issue_gather(i, buf, sem):
        for j in range(GATHERS):
            pltpu.async_copy(
                table_hbm.at[idx_v.at[i * GATHERS + j]],
                rows[buf].at[pl.ds(j * HALF, HALF)],
                sem,
            )

    def wait_gather(buf, sem):
        pltpu.make_async_copy(
            out_hbm.at[pl.ds(0, ROWS_PER_CHUNK)], rows[buf], sem
        ).wait()

    def issue_out(i, buf, sem):
        pltpu.async_copy(
            rows[buf],
            out_hbm.at[pl.ds((chunk0 + i) * ROWS_PER_CHUNK, ROWS_PER_CHUNK)],
            sem,
        )

    def wait_out(buf, sem):
        pltpu.make_async_copy(
            rows[buf], out_hbm.at[pl.ds(0, ROWS_PER_CHUNK)], sem
        ).wait()

    def add_pe(buf):
        def body(r, _):
            for c in range(DIM // 16):
                sl = pl.ds(c * 16, 16)
                pe_c = pe_v[r, sl]
                for s in range(SEQ_PER_CHUNK):
                    plsc.addupdate(rows[buf].at[s * S + r, sl], pe_c)
            return 0
        lax.fori_loop(0, S, body, 0)

    issue_gather(0, 0, gsem0)

    def chunk_body(i2, _):
        for b in range(2):
            i = i2 * 2 + b
            @pl.when(i >= 1)
            def _():
                wait_out(1 - b, osem[1 - b])

            @pl.when(i < CHUNKS - 1)
            def _():
                issue_gather(i + 1, 1 - b, gsem[1 - b])

            wait_gather(b, gsem[b])
            add_pe(b)
            issue_out(i, b, osem[b])
        return 0

    lax.fori_loop(0, CHUNKS // 2, chunk_body, 0)
    wait_out(1, osem1)


def kernel(input, tok_table):
    idx = input.astype(jnp.int32).reshape(B * 2, HALF)
    out = _emb(idx, tok_table, jnp.asarray(_PE))
    return out.reshape(B, S, DIM)
